# baseline (speedup 1.0000x reference)
import functools

import jax
import jax.numpy as jnp
from jax import lax
from jax.experimental import pallas as pl
from jax.experimental.pallas import tpu as pltpu
from jax.experimental.pallas import tpu_sc as plsc

NC = 2
NS = 16
NW = NC * NS
CH = 128
BR = 128


def _sc_aggr(x_pad, src2, dst2, ew2, zeros_hbm, n_pad, f):
    nchunks = src2.shape[0]
    rpt = n_pad // NS
    nvr = f // 16
    SCN = 8
    ncw1 = (nchunks // NS) // 4
    ncw1 = (ncw1 // SCN) * SCN
    ncw0 = nchunks // NS - ncw1
    assert ncw1 % SCN == 0 and ncw0 > 0
    mesh = plsc.VectorSubcoreMesh(core_axis_name="c", subcore_axis_name="s")

    @functools.partial(
        pl.kernel,
        mesh=mesh,
        out_type=jax.ShapeDtypeStruct((NC * n_pad, f), jnp.float32),
        scratch_types=[
            pltpu.VMEM((SCN, CH), jnp.int32),
            pltpu.VMEM((SCN, CH), jnp.int32),
            pltpu.VMEM((SCN, CH), jnp.float32),
            pltpu.VMEM((2, CH, f), jnp.float32),
            pltpu.VMEM_SHARED((n_pad, f), jnp.float32),
            pltpu.SemaphoreType.DMA,
            pltpu.SemaphoreType.DMA,
            pltpu.SemaphoreType.DMA,
            pltpu.SemaphoreType.DMA,
        ],
    )
    def k(x_hbm, src_hbm, dst_hbm, ew_hbm, z_hbm, out_hbm,
          src_v, dst_v, ew_v, rows_v, acc, sem0, sem1, sem2, sem3):
        c = lax.axis_index("c")
        s = lax.axis_index("s")
        pltpu.sync_copy(z_hbm, acc.at[pl.ds(s * rpt, rpt)])
        base = jnp.where(c == 0, s * ncw0, NS * ncw0 + s * ncw1)
        nsc = jnp.where(c == 0, ncw0 // SCN, ncw1 // SCN)
        plsc.subcore_barrier()

        sems = ((sem0, sem1), (sem2, sem3))
        hc = CH // 2

        def start_gather(g, b):
            pltpu.async_copy(x_hbm.at[src_v.at[g, pl.ds(0, hc)]],
                             rows_v.at[b, pl.ds(0, hc)], sems[b][0])
            pltpu.async_copy(x_hbm.at[src_v.at[g, pl.ds(hc, hc)]],
                             rows_v.at[b, pl.ds(hc, hc)], sems[b][1])

        def wait_gather(g, b):
            pltpu.make_async_copy(x_hbm.at[src_v.at[g, pl.ds(0, hc)]],
                                  rows_v.at[b, pl.ds(0, hc)], sems[b][0]).wait()
            pltpu.make_async_copy(x_hbm.at[src_v.at[g, pl.ds(hc, hc)]],
                                  rows_v.at[b, pl.ds(hc, hc)], sems[b][1]).wait()

        def superchunk(t, carry):
            sbase = base + t * SCN
            pltpu.sync_copy(src_hbm.at[pl.ds(sbase, SCN)], src_v)
            pltpu.sync_copy(dst_hbm.at[pl.ds(sbase, SCN)], dst_v)
            pltpu.sync_copy(ew_hbm.at[pl.ds(sbase, SCN)], ew_v)
            for b in range(2):
                start_gather(b, b)

            def pair(g0, carry1):
                for b in range(2):
                    g = g0 + b
                    wait_gather(g, b)

                    def grp_body(q, carry2):
                        ewg = ew_v[g, pl.ds(q * 16, 16)]
                        for r16 in range(16):
                            ewb = ewg.at[jnp.full((16,), r16, jnp.int32)].get(
                                mode="promise_in_bounds")
                            r = q * 16 + r16
                            for j in range(nvr):
                                v = rows_v[b, r, pl.ds(j * 16, 16)]
                                rows_v[b, r, pl.ds(j * 16, 16)] = v * ewb
                        return carry2

                    lax.fori_loop(0, CH // 16, grp_body, 0)
                    pltpu.sync_copy(rows_v.at[b], acc.at[dst_v.at[g]],
                                    add=True)
                    @pl.when(g + 2 < SCN)
                    def _():
                        start_gather(g + 2, b)
                return carry1

            lax.fori_loop(0, SCN // 2, lambda i, cr: pair(i * 2, cr), 0)
            return carry

        lax.fori_loop(0, nsc, superchunk, 0)
        plsc.subcore_barrier()

        def spin(i, v):
            rows_v[0, 0, pl.ds(0, 16)] = jnp.full((16,), v, jnp.float32)
            return v + 1.0

        lax.fori_loop(0, 512, spin, 0.0)
        plsc.subcore_barrier()
        pltpu.sync_copy(acc.at[pl.ds(s * rpt, rpt)],
                        out_hbm.at[pl.ds(c * n_pad + s * rpt, rpt)])

    out = k(x_pad, src2, dst2, ew2, zeros_hbm)
    return out.reshape(NC, n_pad, f)


def _tc_combine(p, xin, Wrel, brel, Wroot, n_real, want_stats):
    n_pad = xin.shape[0]
    fr, fo = Wroot.shape
    fin = Wrel.shape[0] if Wrel is not None else fo
    grid = (n_pad // BR,)

    def body(*refs):
        if Wrel is not None:
            p_ref, x_ref, wrel_ref, brel_ref, wroot_ref, z_ref, *stat_refs = refs
        else:
            p_ref, x_ref, brel_ref, wroot_ref, z_ref, *stat_refs = refs
        i = pl.program_id(0)
        aggr = p_ref[0] + p_ref[1]
        if Wrel is not None:
            aggr = jnp.dot(aggr, wrel_ref[...],
                           preferred_element_type=jnp.float32)
        z = (aggr
             + jnp.dot(x_ref[...], wroot_ref[...],
                       preferred_element_type=jnp.float32)
             + brel_ref[...])
        z_ref[...] = z
        if want_stats:
            s1_ref, s2_ref = stat_refs
            rows = i * BR + lax.broadcasted_iota(jnp.int32, (BR, 1), 0)
            zm = jnp.where(rows < n_real, z, 0.0)
            s1 = jnp.sum(zm, axis=0, keepdims=True)
            s2 = jnp.sum(zm * zm, axis=0, keepdims=True)

            @pl.when(i == 0)
            def _():
                s1_ref[...] = s1
                s2_ref[...] = s2

            @pl.when(i > 0)
            def _():
                s1_ref[...] += s1
                s2_ref[...] += s2

    out_shape = [jax.ShapeDtypeStruct((n_pad, fo), jnp.float32)]
    out_specs = [pl.BlockSpec((BR, fo), lambda i: (i, 0))]
    if want_stats:
        out_shape += [jax.ShapeDtypeStruct((1, fo), jnp.float32)] * 2
        out_specs += [pl.BlockSpec((1, fo), lambda i: (0, 0)),
                      pl.BlockSpec((1, fo), lambda i: (0, 0))]

    in_specs = [pl.BlockSpec((2, BR, fin), lambda i: (0, i, 0)),
                pl.BlockSpec((BR, fr), lambda i: (i, 0))]
    args = [p, xin]
    if Wrel is not None:
        in_specs.append(pl.BlockSpec((fin, fo), lambda i: (0, 0)))
        args.append(Wrel)
    in_specs += [pl.BlockSpec((1, fo), lambda i: (0, 0)),
                 pl.BlockSpec((fr, fo), lambda i: (0, 0))]
    args += [brel.reshape(1, -1), Wroot]
    return pl.pallas_call(
        body,
        grid=grid,
        in_specs=in_specs,
        out_specs=out_specs if want_stats else out_specs[0],
        out_shape=out_shape if want_stats else out_shape[0],
    )(*args)


def _tc_norm_relu(z, s1, s2, g, b, n_real):
    n_pad, f = z.shape
    inv_n = 1.0 / float(n_real)

    def body(z_ref, s1_ref, s2_ref, g_ref, b_ref, o_ref):
        m = s1_ref[...] * inv_n
        v = s2_ref[...] * inv_n - m * m
        sc = g_ref[...] * lax.rsqrt(v + 1e-5)
        o_ref[...] = jnp.maximum((z_ref[...] - m) * sc + b_ref[...], 0.0)

    return pl.pallas_call(
        body,
        grid=(n_pad // BR,),
        in_specs=[
            pl.BlockSpec((BR, f), lambda i: (i, 0)),
            pl.BlockSpec((1, f), lambda i: (0, 0)),
            pl.BlockSpec((1, f), lambda i: (0, 0)),
            pl.BlockSpec((1, f), lambda i: (0, 0)),
            pl.BlockSpec((1, f), lambda i: (0, 0)),
        ],
        out_specs=pl.BlockSpec((BR, f), lambda i: (i, 0)),
        out_shape=jax.ShapeDtypeStruct((n_pad, f), jnp.float32),
    )(z, s1, s2, g.reshape(1, -1), b.reshape(1, -1))


def _tc_enc1_cnn(p, xin, Wrel, brel, Wroot, W0, b0, c1, W1, W2, b2, c3, W3,
                 Wnext):
    n_pad = xin.shape[0]
    fin, fo = Wrel.shape
    fr = Wroot.shape[0]
    fn2 = Wnext.shape[1]

    def body(p_ref, x_ref, wrel_ref, brel_ref, wroot_ref,
             w0_ref, b0_ref, c1_ref, w1_ref, w2_ref, b2_ref, c3_ref, w3_ref,
             wn_ref, o_ref, o2_ref):
        aggr = p_ref[0] + p_ref[1]
        z2 = (jnp.dot(aggr, wrel_ref[...], preferred_element_type=jnp.float32)
              + jnp.dot(x_ref[...], wroot_ref[...],
                        preferred_element_type=jnp.float32)
              + brel_ref[...])
        a = jnp.maximum(
            jnp.dot(z2, w0_ref[...], preferred_element_type=jnp.float32)
            + b0_ref[...], 0.0)
        t = (jnp.dot(a, w1_ref[...], preferred_element_type=jnp.float32)
             + c1_ref[0, 0])
        d = jnp.maximum(
            jnp.dot(t, w2_ref[...], preferred_element_type=jnp.float32)
            + b2_ref[...], 0.0)
        zd = (jnp.dot(d, w3_ref[...], preferred_element_type=jnp.float32)
              + c3_ref[0, 0])
        o_ref[...] = zd
        o2_ref[...] = jnp.dot(zd, wn_ref[...],
                              preferred_element_type=jnp.float32)

    full = lambda shape: pl.BlockSpec(shape, lambda i: tuple(0 for _ in shape))
    return pl.pallas_call(
        body,
        grid=(n_pad // BR,),
        in_specs=[
            pl.BlockSpec((2, BR, fin), lambda i: (0, i, 0)),
            pl.BlockSpec((BR, fr), lambda i: (i, 0)),
            full((fin, fo)),
            full((1, fo)),
            full((fr, fo)),
            full((64, 256)),
            full((1, 256)),
            full((1, 1)),
            full((256, 64)),
            full((64, 256)),
            full((1, 256)),
            full((1, 1)),
            full((256, 64)),
            full((64, fn2)),
        ],
        out_specs=[pl.BlockSpec((BR, 64), lambda i: (i, 0)),
                   pl.BlockSpec((BR, fn2), lambda i: (i, 0))],
        out_shape=[jax.ShapeDtypeStruct((n_pad, 64), jnp.float32),
                   jax.ShapeDtypeStruct((n_pad, fn2), jnp.float32)],
    )(p, xin, Wrel, brel.reshape(1, -1), Wroot,
      W0, b0.reshape(1, -1), c1.reshape(1, 1), W1,
      W2, b2.reshape(1, -1), c3.reshape(1, 1), W3, Wnext)


def _band(w3):
    eye = functools.partial(jnp.eye, 64, dtype=jnp.float32)
    return eye(k=1) * w3[0] + eye() * w3[1] + eye(k=-1) * w3[2]


def kernel(x1, x2, x3, x4, x5, x6, x7, edge_index, edge_attr,
           enc0_Wrel, enc0_brel, enc0_Wroot, enc1_Wrel, enc1_brel, enc1_Wroot,
           ebn0_g, ebn0_b,
           dec0_Wrel, dec0_brel, dec0_Wroot, dec1_Wrel, dec1_brel, dec1_Wroot,
           dbn0_g, dbn0_b,
           ecw0, ecb0, ecw1, ecb1, dcw0, dcb0, dcw1, dcb1):
    b, f = x1.shape
    n = 7 * b
    n_pad = -(-n // BR) * BR
    h = enc1_Wrel.shape[1]

    x = jnp.stack([x1, x2, x3, x4, x5, x6, x7], axis=1).reshape(n, f)
    x_pad = jnp.zeros((n_pad, f), jnp.float32).at[:n].set(x)

    src = edge_index[0]
    dst = edge_index[1]
    e = src.shape[0]
    ncw = -(-e // (NW * CH))
    ncw = -(-ncw // 16) * 16
    e_pad = NW * CH * ncw
    pad = e_pad - e
    src2 = jnp.concatenate([src, jnp.zeros((pad,), jnp.int32)]).reshape(-1, CH)
    dst2 = jnp.concatenate([dst, jnp.full((pad,), n_pad - 1, jnp.int32)]
                           ).reshape(-1, CH)
    ew2 = jnp.concatenate([edge_attr, jnp.zeros((pad,), jnp.float32)]
                          ).reshape(-1, CH)
    zeros_f = jnp.zeros((n_pad // NS, f), jnp.float32)

    W0 = jnp.concatenate([_band(ecw0[c, 0]) for c in range(4)], axis=1)
    b0 = jnp.repeat(ecb0, 64)
    W1 = jnp.concatenate([_band(ecw1[0, c]) for c in range(4)], axis=0)
    w2t = jnp.flip(dcw0, axis=2).transpose(1, 0, 2)
    W2 = jnp.concatenate([_band(w2t[c, 0]) for c in range(4)], axis=1)
    b2 = jnp.repeat(dcb0, 64)
    w3t = jnp.flip(dcw1, axis=2).transpose(1, 0, 2)
    W3 = jnp.concatenate([_band(w3t[0, c]) for c in range(4)], axis=0)

    pa = _sc_aggr(x_pad, src2, dst2, ew2, zeros_f, n_pad, f)
    z1, s1, s2 = _tc_combine(pa, x_pad, enc0_Wrel, enc0_brel, enc0_Wroot,
                             n, True)
    z1bn = _tc_norm_relu(z1, s1, s2, ebn0_g, ebn0_b, n)
    pb = _sc_aggr(z1bn, src2, dst2, ew2, zeros_f, n_pad, f)
    zd, zd2 = _tc_enc1_cnn(pb, z1bn, enc1_Wrel, enc1_brel, enc1_Wroot,
                           W0, b0, ecb1, W1, W2, b2, dcb1, W3, dec0_Wrel)
    pc = _sc_aggr(zd2, src2, dst2, ew2, zeros_f, n_pad, f)
    h1, t1, t2 = _tc_combine(pc, zd, None, dec0_brel, dec0_Wroot,
                             n, True)
    h1bn = _tc_norm_relu(h1, t1, t2, dbn0_g, dbn0_b, n)
    pd = _sc_aggr(h1bn, src2, dst2, ew2, zeros_f, n_pad, f)
    hh = _tc_combine(pd, h1bn, dec1_Wrel, dec1_brel, dec1_Wroot, n, False)

    x_out = jnp.concatenate([x1, x2, x3, x4, x5, x6, x7], axis=1)
    h_out = hh[:n].reshape(b, 7 * f)
    return (x_out, h_out)


# speedup vs baseline: 1.0487x; 1.0487x over previous
import functools

import jax
import jax.numpy as jnp
from jax import lax
from jax.experimental import pallas as pl
from jax.experimental.pallas import tpu as pltpu
from jax.experimental.pallas import tpu_sc as plsc

NC = 2
NS = 16
NW = NC * NS
CH = 128
BR = 128


def _sc_aggr(x_pad, src2, dst2, ew2, zeros_hbm, n_pad, f):
    nchunks = src2.shape[0]
    rpt = n_pad // NS
    nvr = f // 16
    SCN = 8
    ncw1 = (nchunks // NS) * 3 // 20
    ncw1 = (ncw1 // SCN) * SCN
    ncw0 = nchunks // NS - ncw1
    assert ncw1 % SCN == 0 and ncw0 > 0
    mesh = plsc.VectorSubcoreMesh(core_axis_name="c", subcore_axis_name="s")

    @functools.partial(
        pl.kernel,
        mesh=mesh,
        out_type=jax.ShapeDtypeStruct((NC * n_pad, f), jnp.float32),
        scratch_types=[
            pltpu.VMEM((SCN, CH), jnp.int32),
            pltpu.VMEM((SCN, CH), jnp.int32),
            pltpu.VMEM((SCN, CH), jnp.float32),
            pltpu.VMEM((2, CH, f), jnp.float32),
            pltpu.VMEM_SHARED((n_pad, f), jnp.float32),
            pltpu.SemaphoreType.DMA,
            pltpu.SemaphoreType.DMA,
            pltpu.SemaphoreType.DMA,
            pltpu.SemaphoreType.DMA,
        ],
    )
    def k(x_hbm, src_hbm, dst_hbm, ew_hbm, z_hbm, out_hbm,
          src_v, dst_v, ew_v, rows_v, acc, sem0, sem1, sem2, sem3):
        c = lax.axis_index("c")
        s = lax.axis_index("s")
        pltpu.sync_copy(z_hbm, acc.at[pl.ds(s * rpt, rpt)])
        base = jnp.where(c == 0, s * ncw0, NS * ncw0 + s * ncw1)
        nsc = jnp.where(c == 0, ncw0 // SCN, ncw1 // SCN)
        plsc.subcore_barrier()

        sems = ((sem0, sem1), (sem2, sem3))
        hc = CH // 2

        def start_gather(g, b):
            pltpu.async_copy(x_hbm.at[src_v.at[g, pl.ds(0, hc)]],
                             rows_v.at[b, pl.ds(0, hc)], sems[b][0])
            pltpu.async_copy(x_hbm.at[src_v.at[g, pl.ds(hc, hc)]],
                             rows_v.at[b, pl.ds(hc, hc)], sems[b][1])

        def wait_gather(g, b):
            pltpu.make_async_copy(x_hbm.at[src_v.at[g, pl.ds(0, hc)]],
                                  rows_v.at[b, pl.ds(0, hc)], sems[b][0]).wait()
            pltpu.make_async_copy(x_hbm.at[src_v.at[g, pl.ds(hc, hc)]],
                                  rows_v.at[b, pl.ds(hc, hc)], sems[b][1]).wait()

        def superchunk(t, carry):
            sbase = base + t * SCN
            pltpu.sync_copy(src_hbm.at[pl.ds(sbase, SCN)], src_v)
            pltpu.sync_copy(dst_hbm.at[pl.ds(sbase, SCN)], dst_v)
            pltpu.sync_copy(ew_hbm.at[pl.ds(sbase, SCN)], ew_v)
            for b in range(2):
                start_gather(b, b)

            def pair(g0, carry1):
                for b in range(2):
                    g = g0 + b
                    wait_gather(g, b)

                    def grp_body(q, carry2):
                        ewg = ew_v[g, pl.ds(q * 16, 16)]
                        for r16 in range(16):
                            ewb = ewg.at[jnp.full((16,), r16, jnp.int32)].get(
                                mode="promise_in_bounds")
                            r = q * 16 + r16
                            for j in range(nvr):
                                v = rows_v[b, r, pl.ds(j * 16, 16)]
                                rows_v[b, r, pl.ds(j * 16, 16)] = v * ewb
                        return carry2

                    lax.fori_loop(0, CH // 16, grp_body, 0)
                    pltpu.sync_copy(rows_v.at[b], acc.at[dst_v.at[g]],
                                    add=True)
                    @pl.when(g + 2 < SCN)
                    def _():
                        start_gather(g + 2, b)
                return carry1

            lax.fori_loop(0, SCN // 2, lambda i, cr: pair(i * 2, cr), 0)
            return carry

        lax.fori_loop(0, nsc, superchunk, 0)
        plsc.subcore_barrier()

        def spin(i, v):
            rows_v[0, 0, pl.ds(0, 16)] = jnp.full((16,), v, jnp.float32)
            return v + 1.0

        lax.fori_loop(0, 512, spin, 0.0)
        plsc.subcore_barrier()
        pltpu.sync_copy(acc.at[pl.ds(s * rpt, rpt)],
                        out_hbm.at[pl.ds(c * n_pad + s * rpt, rpt)])

    out = k(x_pad, src2, dst2, ew2, zeros_hbm)
    return out.reshape(NC, n_pad, f)


def _tc_combine(p, xin, Wrel, brel, Wroot, n_real, want_stats):
    n_pad = xin.shape[0]
    fr, fo = Wroot.shape
    fin = Wrel.shape[0] if Wrel is not None else fo
    grid = (n_pad // BR,)

    def body(*refs):
        if Wrel is not None:
            p_ref, x_ref, wrel_ref, brel_ref, wroot_ref, z_ref, *stat_refs = refs
        else:
            p_ref, x_ref, brel_ref, wroot_ref, z_ref, *stat_refs = refs
        i = pl.program_id(0)
        aggr = p_ref[0] + p_ref[1]
        if Wrel is not None:
            aggr = jnp.dot(aggr, wrel_ref[...],
                           preferred_element_type=jnp.float32)
        z = (aggr
             + jnp.dot(x_ref[...], wroot_ref[...],
                       preferred_element_type=jnp.float32)
             + brel_ref[...])
        z_ref[...] = z
        if want_stats:
            s1_ref, s2_ref = stat_refs
            rows = i * BR + lax.broadcasted_iota(jnp.int32, (BR, 1), 0)
            zm = jnp.where(rows < n_real, z, 0.0)
            s1 = jnp.sum(zm, axis=0, keepdims=True)
            s2 = jnp.sum(zm * zm, axis=0, keepdims=True)

            @pl.when(i == 0)
            def _():
                s1_ref[...] = s1
                s2_ref[...] = s2

            @pl.when(i > 0)
            def _():
                s1_ref[...] += s1
                s2_ref[...] += s2

    out_shape = [jax.ShapeDtypeStruct((n_pad, fo), jnp.float32)]
    out_specs = [pl.BlockSpec((BR, fo), lambda i: (i, 0))]
    if want_stats:
        out_shape += [jax.ShapeDtypeStruct((1, fo), jnp.float32)] * 2
        out_specs += [pl.BlockSpec((1, fo), lambda i: (0, 0)),
                      pl.BlockSpec((1, fo), lambda i: (0, 0))]

    in_specs = [pl.BlockSpec((2, BR, fin), lambda i: (0, i, 0)),
                pl.BlockSpec((BR, fr), lambda i: (i, 0))]
    args = [p, xin]
    if Wrel is not None:
        in_specs.append(pl.BlockSpec((fin, fo), lambda i: (0, 0)))
        args.append(Wrel)
    in_specs += [pl.BlockSpec((1, fo), lambda i: (0, 0)),
                 pl.BlockSpec((fr, fo), lambda i: (0, 0))]
    args += [brel.reshape(1, -1), Wroot]
    return pl.pallas_call(
        body,
        grid=grid,
        in_specs=in_specs,
        out_specs=out_specs if want_stats else out_specs[0],
        out_shape=out_shape if want_stats else out_shape[0],
    )(*args)


def _tc_norm_relu(z, s1, s2, g, b, n_real):
    n_pad, f = z.shape
    inv_n = 1.0 / float(n_real)

    def body(z_ref, s1_ref, s2_ref, g_ref, b_ref, o_ref):
        m = s1_ref[...] * inv_n
        v = s2_ref[...] * inv_n - m * m
        sc = g_ref[...] * lax.rsqrt(v + 1e-5)
        o_ref[...] = jnp.maximum((z_ref[...] - m) * sc + b_ref[...], 0.0)

    return pl.pallas_call(
        body,
        grid=(n_pad // BR,),
        in_specs=[
            pl.BlockSpec((BR, f), lambda i: (i, 0)),
            pl.BlockSpec((1, f), lambda i: (0, 0)),
            pl.BlockSpec((1, f), lambda i: (0, 0)),
            pl.BlockSpec((1, f), lambda i: (0, 0)),
            pl.BlockSpec((1, f), lambda i: (0, 0)),
        ],
        out_specs=pl.BlockSpec((BR, f), lambda i: (i, 0)),
        out_shape=jax.ShapeDtypeStruct((n_pad, f), jnp.float32),
    )(z, s1, s2, g.reshape(1, -1), b.reshape(1, -1))


def _tc_enc1_cnn(p, xin, Wrel, brel, Wroot, W0, b0, c1, W1, W2, b2, c3, W3,
                 Wnext):
    n_pad = xin.shape[0]
    fin, fo = Wrel.shape
    fr = Wroot.shape[0]
    fn2 = Wnext.shape[1]

    def body(p_ref, x_ref, wrel_ref, brel_ref, wroot_ref,
             w0_ref, b0_ref, c1_ref, w1_ref, w2_ref, b2_ref, c3_ref, w3_ref,
             wn_ref, o_ref, o2_ref):
        aggr = p_ref[0] + p_ref[1]
        z2 = (jnp.dot(aggr, wrel_ref[...], preferred_element_type=jnp.float32)
              + jnp.dot(x_ref[...], wroot_ref[...],
                        preferred_element_type=jnp.float32)
              + brel_ref[...])
        a = jnp.maximum(
            jnp.dot(z2, w0_ref[...], preferred_element_type=jnp.float32)
            + b0_ref[...], 0.0)
        t = (jnp.dot(a, w1_ref[...], preferred_element_type=jnp.float32)
             + c1_ref[0, 0])
        d = jnp.maximum(
            jnp.dot(t, w2_ref[...], preferred_element_type=jnp.float32)
            + b2_ref[...], 0.0)
        zd = (jnp.dot(d, w3_ref[...], preferred_element_type=jnp.float32)
              + c3_ref[0, 0])
        o_ref[...] = zd
        o2_ref[...] = jnp.dot(zd, wn_ref[...],
                              preferred_element_type=jnp.float32)

    full = lambda shape: pl.BlockSpec(shape, lambda i: tuple(0 for _ in shape))
    return pl.pallas_call(
        body,
        grid=(n_pad // BR,),
        in_specs=[
            pl.BlockSpec((2, BR, fin), lambda i: (0, i, 0)),
            pl.BlockSpec((BR, fr), lambda i: (i, 0)),
            full((fin, fo)),
            full((1, fo)),
            full((fr, fo)),
            full((64, 256)),
            full((1, 256)),
            full((1, 1)),
            full((256, 64)),
            full((64, 256)),
            full((1, 256)),
            full((1, 1)),
            full((256, 64)),
            full((64, fn2)),
        ],
        out_specs=[pl.BlockSpec((BR, 64), lambda i: (i, 0)),
                   pl.BlockSpec((BR, fn2), lambda i: (i, 0))],
        out_shape=[jax.ShapeDtypeStruct((n_pad, 64), jnp.float32),
                   jax.ShapeDtypeStruct((n_pad, fn2), jnp.float32)],
    )(p, xin, Wrel, brel.reshape(1, -1), Wroot,
      W0, b0.reshape(1, -1), c1.reshape(1, 1), W1,
      W2, b2.reshape(1, -1), c3.reshape(1, 1), W3, Wnext)


def _band(w3):
    eye = functools.partial(jnp.eye, 64, dtype=jnp.float32)
    return eye(k=1) * w3[0] + eye() * w3[1] + eye(k=-1) * w3[2]


def kernel(x1, x2, x3, x4, x5, x6, x7, edge_index, edge_attr,
           enc0_Wrel, enc0_brel, enc0_Wroot, enc1_Wrel, enc1_brel, enc1_Wroot,
           ebn0_g, ebn0_b,
           dec0_Wrel, dec0_brel, dec0_Wroot, dec1_Wrel, dec1_brel, dec1_Wroot,
           dbn0_g, dbn0_b,
           ecw0, ecb0, ecw1, ecb1, dcw0, dcb0, dcw1, dcb1):
    b, f = x1.shape
    n = 7 * b
    n_pad = -(-n // BR) * BR
    h = enc1_Wrel.shape[1]

    x = jnp.stack([x1, x2, x3, x4, x5, x6, x7], axis=1).reshape(n, f)
    x_pad = jnp.zeros((n_pad, f), jnp.float32).at[:n].set(x)

    src = edge_index[0]
    dst = edge_index[1]
    e = src.shape[0]
    ncw = -(-e // (NW * CH))
    ncw = -(-ncw // 16) * 16
    e_pad = NW * CH * ncw
    pad = e_pad - e
    src2 = jnp.concatenate([src, jnp.zeros((pad,), jnp.int32)]).reshape(-1, CH)
    dst2 = jnp.concatenate([dst, jnp.full((pad,), n_pad - 1, jnp.int32)]
                           ).reshape(-1, CH)
    ew2 = jnp.concatenate([edge_attr, jnp.zeros((pad,), jnp.float32)]
                          ).reshape(-1, CH)
    zeros_f = jnp.zeros((n_pad // NS, f), jnp.float32)

    W0 = jnp.concatenate([_band(ecw0[c, 0]) for c in range(4)], axis=1)
    b0 = jnp.repeat(ecb0, 64)
    W1 = jnp.concatenate([_band(ecw1[0, c]) for c in range(4)], axis=0)
    w2t = jnp.flip(dcw0, axis=2).transpose(1, 0, 2)
    W2 = jnp.concatenate([_band(w2t[c, 0]) for c in range(4)], axis=1)
    b2 = jnp.repeat(dcb0, 64)
    w3t = jnp.flip(dcw1, axis=2).transpose(1, 0, 2)
    W3 = jnp.concatenate([_band(w3t[0, c]) for c in range(4)], axis=0)

    pa = _sc_aggr(x_pad, src2, dst2, ew2, zeros_f, n_pad, f)
    z1, s1, s2 = _tc_combine(pa, x_pad, enc0_Wrel, enc0_brel, enc0_Wroot,
                             n, True)
    z1bn = _tc_norm_relu(z1, s1, s2, ebn0_g, ebn0_b, n)
    pb = _sc_aggr(z1bn, src2, dst2, ew2, zeros_f, n_pad, f)
    zd, zd2 = _tc_enc1_cnn(pb, z1bn, enc1_Wrel, enc1_brel, enc1_Wroot,
                           W0, b0, ecb1, W1, W2, b2, dcb1, W3, dec0_Wrel)
    pc = _sc_aggr(zd2, src2, dst2, ew2, zeros_f, n_pad, f)
    h1, t1, t2 = _tc_combine(pc, zd, None, dec0_brel, dec0_Wroot,
                             n, True)
    h1bn = _tc_norm_relu(h1, t1, t2, dbn0_g, dbn0_b, n)
    pd = _sc_aggr(h1bn, src2, dst2, ew2, zeros_f, n_pad, f)
    hh = _tc_combine(pd, h1bn, dec1_Wrel, dec1_brel, dec1_Wroot, n, False)

    x_out = jnp.concatenate([x1, x2, x3, x4, x5, x6, x7], axis=1)
    h_out = hh[:n].reshape(b, 7 * f)
    return (x_out, h_out)


# speedup vs baseline: 1.0933x; 1.0426x over previous
import functools

import jax
import jax.numpy as jnp
from jax import lax
from jax.experimental import pallas as pl
from jax.experimental.pallas import tpu as pltpu
from jax.experimental.pallas import tpu_sc as plsc

NC = 2
NS = 16
NW = NC * NS
CH = 128
BR = 128


def _sc_aggr(x_pad, src2, dst2, ew2, zeros_hbm, n_pad, f):
    nchunks = src2.shape[0]
    rpt = n_pad // NS
    nvr = f // 16
    SCN = 8
    ncw1 = (nchunks // NS) // 20
    ncw1 = (ncw1 // SCN) * SCN
    ncw0 = nchunks // NS - ncw1
    assert ncw1 % SCN == 0 and ncw0 > 0
    mesh = plsc.VectorSubcoreMesh(core_axis_name="c", subcore_axis_name="s")

    @functools.partial(
        pl.kernel,
        mesh=mesh,
        out_type=jax.ShapeDtypeStruct((NC * n_pad, f), jnp.float32),
        scratch_types=[
            pltpu.VMEM((SCN, CH), jnp.int32),
            pltpu.VMEM((SCN, CH), jnp.int32),
            pltpu.VMEM((SCN, CH), jnp.float32),
            pltpu.VMEM((2, CH, f), jnp.float32),
            pltpu.VMEM_SHARED((n_pad, f), jnp.float32),
            pltpu.SemaphoreType.DMA,
            pltpu.SemaphoreType.DMA,
            pltpu.SemaphoreType.DMA,
            pltpu.SemaphoreType.DMA,
        ],
    )
    def k(x_hbm, src_hbm, dst_hbm, ew_hbm, z_hbm, out_hbm,
          src_v, dst_v, ew_v, rows_v, acc, sem0, sem1, sem2, sem3):
        c = lax.axis_index("c")
        s = lax.axis_index("s")
        pltpu.sync_copy(z_hbm, acc.at[pl.ds(s * rpt, rpt)])
        base = jnp.where(c == 0, s * ncw0, NS * ncw0 + s * ncw1)
        nsc = jnp.where(c == 0, ncw0 // SCN, ncw1 // SCN)
        plsc.subcore_barrier()

        sems = ((sem0, sem1), (sem2, sem3))
        hc = CH // 2

        def start_gather(g, b):
            pltpu.async_copy(x_hbm.at[src_v.at[g, pl.ds(0, hc)]],
                             rows_v.at[b, pl.ds(0, hc)], sems[b][0])
            pltpu.async_copy(x_hbm.at[src_v.at[g, pl.ds(hc, hc)]],
                             rows_v.at[b, pl.ds(hc, hc)], sems[b][1])

        def wait_gather(g, b):
            pltpu.make_async_copy(x_hbm.at[src_v.at[g, pl.ds(0, hc)]],
                                  rows_v.at[b, pl.ds(0, hc)], sems[b][0]).wait()
            pltpu.make_async_copy(x_hbm.at[src_v.at[g, pl.ds(hc, hc)]],
                                  rows_v.at[b, pl.ds(hc, hc)], sems[b][1]).wait()

        def superchunk(t, carry):
            sbase = base + t * SCN
            pltpu.sync_copy(src_hbm.at[pl.ds(sbase, SCN)], src_v)
            pltpu.sync_copy(dst_hbm.at[pl.ds(sbase, SCN)], dst_v)
            pltpu.sync_copy(ew_hbm.at[pl.ds(sbase, SCN)], ew_v)
            for b in range(2):
                start_gather(b, b)

            def pair(g0, carry1):
                for b in range(2):
                    g = g0 + b
                    wait_gather(g, b)

                    def grp_body(q, carry2):
                        ewg = ew_v[g, pl.ds(q * 16, 16)]
                        for r16 in range(16):
                            ewb = ewg.at[jnp.full((16,), r16, jnp.int32)].get(
                                mode="promise_in_bounds")
                            r = q * 16 + r16
                            for j in range(nvr):
                                v = rows_v[b, r, pl.ds(j * 16, 16)]
                                rows_v[b, r, pl.ds(j * 16, 16)] = v * ewb
                        return carry2

                    lax.fori_loop(0, CH // 16, grp_body, 0)
                    pltpu.sync_copy(rows_v.at[b], acc.at[dst_v.at[g]],
                                    add=True)
                    @pl.when(g + 2 < SCN)
                    def _():
                        start_gather(g + 2, b)
                return carry1

            lax.fori_loop(0, SCN // 2, lambda i, cr: pair(i * 2, cr), 0)
            return carry

        lax.fori_loop(0, nsc, superchunk, 0)
        plsc.subcore_barrier()

        def spin(i, v):
            rows_v[0, 0, pl.ds(0, 16)] = jnp.full((16,), v, jnp.float32)
            return v + 1.0

        lax.fori_loop(0, 512, spin, 0.0)
        plsc.subcore_barrier()
        pltpu.sync_copy(acc.at[pl.ds(s * rpt, rpt)],
                        out_hbm.at[pl.ds(c * n_pad + s * rpt, rpt)])

    out = k(x_pad, src2, dst2, ew2, zeros_hbm)
    return out.reshape(NC, n_pad, f)


def _tc_combine(p, xin, Wrel, brel, Wroot, n_real, want_stats):
    n_pad = xin.shape[0]
    fr, fo = Wroot.shape
    fin = Wrel.shape[0] if Wrel is not None else fo
    grid = (n_pad // BR,)

    def body(*refs):
        if Wrel is not None:
            p_ref, x_ref, wrel_ref, brel_ref, wroot_ref, z_ref, *stat_refs = refs
        else:
            p_ref, x_ref, brel_ref, wroot_ref, z_ref, *stat_refs = refs
        i = pl.program_id(0)
        aggr = p_ref[0] + p_ref[1]
        if Wrel is not None:
            aggr = jnp.dot(aggr, wrel_ref[...],
                           preferred_element_type=jnp.float32)
        z = (aggr
             + jnp.dot(x_ref[...], wroot_ref[...],
                       preferred_element_type=jnp.float32)
             + brel_ref[...])
        z_ref[...] = z
        if want_stats:
            s1_ref, s2_ref = stat_refs
            rows = i * BR + lax.broadcasted_iota(jnp.int32, (BR, 1), 0)
            zm = jnp.where(rows < n_real, z, 0.0)
            s1 = jnp.sum(zm, axis=0, keepdims=True)
            s2 = jnp.sum(zm * zm, axis=0, keepdims=True)

            @pl.when(i == 0)
            def _():
                s1_ref[...] = s1
                s2_ref[...] = s2

            @pl.when(i > 0)
            def _():
                s1_ref[...] += s1
                s2_ref[...] += s2

    out_shape = [jax.ShapeDtypeStruct((n_pad, fo), jnp.float32)]
    out_specs = [pl.BlockSpec((BR, fo), lambda i: (i, 0))]
    if want_stats:
        out_shape += [jax.ShapeDtypeStruct((1, fo), jnp.float32)] * 2
        out_specs += [pl.BlockSpec((1, fo), lambda i: (0, 0)),
                      pl.BlockSpec((1, fo), lambda i: (0, 0))]

    in_specs = [pl.BlockSpec((2, BR, fin), lambda i: (0, i, 0)),
                pl.BlockSpec((BR, fr), lambda i: (i, 0))]
    args = [p, xin]
    if Wrel is not None:
        in_specs.append(pl.BlockSpec((fin, fo), lambda i: (0, 0)))
        args.append(Wrel)
    in_specs += [pl.BlockSpec((1, fo), lambda i: (0, 0)),
                 pl.BlockSpec((fr, fo), lambda i: (0, 0))]
    args += [brel.reshape(1, -1), Wroot]
    return pl.pallas_call(
        body,
        grid=grid,
        in_specs=in_specs,
        out_specs=out_specs if want_stats else out_specs[0],
        out_shape=out_shape if want_stats else out_shape[0],
    )(*args)


def _tc_norm_relu(z, s1, s2, g, b, n_real):
    n_pad, f = z.shape
    inv_n = 1.0 / float(n_real)

    def body(z_ref, s1_ref, s2_ref, g_ref, b_ref, o_ref):
        m = s1_ref[...] * inv_n
        v = s2_ref[...] * inv_n - m * m
        sc = g_ref[...] * lax.rsqrt(v + 1e-5)
        o_ref[...] = jnp.maximum((z_ref[...] - m) * sc + b_ref[...], 0.0)

    return pl.pallas_call(
        body,
        grid=(n_pad // BR,),
        in_specs=[
            pl.BlockSpec((BR, f), lambda i: (i, 0)),
            pl.BlockSpec((1, f), lambda i: (0, 0)),
            pl.BlockSpec((1, f), lambda i: (0, 0)),
            pl.BlockSpec((1, f), lambda i: (0, 0)),
            pl.BlockSpec((1, f), lambda i: (0, 0)),
        ],
        out_specs=pl.BlockSpec((BR, f), lambda i: (i, 0)),
        out_shape=jax.ShapeDtypeStruct((n_pad, f), jnp.float32),
    )(z, s1, s2, g.reshape(1, -1), b.reshape(1, -1))


def _tc_enc1_cnn(p, xin, Wrel, brel, Wroot, W0, b0, c1, W1, W2, b2, c3, W3,
                 Wnext):
    n_pad = xin.shape[0]
    fin, fo = Wrel.shape
    fr = Wroot.shape[0]
    fn2 = Wnext.shape[1]

    def body(p_ref, x_ref, wrel_ref, brel_ref, wroot_ref,
             w0_ref, b0_ref, c1_ref, w1_ref, w2_ref, b2_ref, c3_ref, w3_ref,
             wn_ref, o_ref, o2_ref):
        aggr = p_ref[0] + p_ref[1]
        z2 = (jnp.dot(aggr, wrel_ref[...], preferred_element_type=jnp.float32)
              + jnp.dot(x_ref[...], wroot_ref[...],
                        preferred_element_type=jnp.float32)
              + brel_ref[...])
        a = jnp.maximum(
            jnp.dot(z2, w0_ref[...], preferred_element_type=jnp.float32)
            + b0_ref[...], 0.0)
        t = (jnp.dot(a, w1_ref[...], preferred_element_type=jnp.float32)
             + c1_ref[0, 0])
        d = jnp.maximum(
            jnp.dot(t, w2_ref[...], preferred_element_type=jnp.float32)
            + b2_ref[...], 0.0)
        zd = (jnp.dot(d, w3_ref[...], preferred_element_type=jnp.float32)
              + c3_ref[0, 0])
        o_ref[...] = zd
        o2_ref[...] = jnp.dot(zd, wn_ref[...],
                              preferred_element_type=jnp.float32)

    full = lambda shape: pl.BlockSpec(shape, lambda i: tuple(0 for _ in shape))
    return pl.pallas_call(
        body,
        grid=(n_pad // BR,),
        in_specs=[
            pl.BlockSpec((2, BR, fin), lambda i: (0, i, 0)),
            pl.BlockSpec((BR, fr), lambda i: (i, 0)),
            full((fin, fo)),
            full((1, fo)),
            full((fr, fo)),
            full((64, 256)),
            full((1, 256)),
            full((1, 1)),
            full((256, 64)),
            full((64, 256)),
            full((1, 256)),
            full((1, 1)),
            full((256, 64)),
            full((64, fn2)),
        ],
        out_specs=[pl.BlockSpec((BR, 64), lambda i: (i, 0)),
                   pl.BlockSpec((BR, fn2), lambda i: (i, 0))],
        out_shape=[jax.ShapeDtypeStruct((n_pad, 64), jnp.float32),
                   jax.ShapeDtypeStruct((n_pad, fn2), jnp.float32)],
    )(p, xin, Wrel, brel.reshape(1, -1), Wroot,
      W0, b0.reshape(1, -1), c1.reshape(1, 1), W1,
      W2, b2.reshape(1, -1), c3.reshape(1, 1), W3, Wnext)


def _band(w3):
    eye = functools.partial(jnp.eye, 64, dtype=jnp.float32)
    return eye(k=1) * w3[0] + eye() * w3[1] + eye(k=-1) * w3[2]


def kernel(x1, x2, x3, x4, x5, x6, x7, edge_index, edge_attr,
           enc0_Wrel, enc0_brel, enc0_Wroot, enc1_Wrel, enc1_brel, enc1_Wroot,
           ebn0_g, ebn0_b,
           dec0_Wrel, dec0_brel, dec0_Wroot, dec1_Wrel, dec1_brel, dec1_Wroot,
           dbn0_g, dbn0_b,
           ecw0, ecb0, ecw1, ecb1, dcw0, dcb0, dcw1, dcb1):
    b, f = x1.shape
    n = 7 * b
    n_pad = -(-n // BR) * BR
    h = enc1_Wrel.shape[1]

    x = jnp.stack([x1, x2, x3, x4, x5, x6, x7], axis=1).reshape(n, f)
    x_pad = jnp.zeros((n_pad, f), jnp.float32).at[:n].set(x)

    src = edge_index[0]
    dst = edge_index[1]
    e = src.shape[0]
    ncw = -(-e // (NW * CH))
    ncw = -(-ncw // 16) * 16
    e_pad = NW * CH * ncw
    pad = e_pad - e
    src2 = jnp.concatenate([src, jnp.zeros((pad,), jnp.int32)]).reshape(-1, CH)
    dst2 = jnp.concatenate([dst, jnp.full((pad,), n_pad - 1, jnp.int32)]
                           ).reshape(-1, CH)
    ew2 = jnp.concatenate([edge_attr, jnp.zeros((pad,), jnp.float32)]
                          ).reshape(-1, CH)
    zeros_f = jnp.zeros((n_pad // NS, f), jnp.float32)

    W0 = jnp.concatenate([_band(ecw0[c, 0]) for c in range(4)], axis=1)
    b0 = jnp.repeat(ecb0, 64)
    W1 = jnp.concatenate([_band(ecw1[0, c]) for c in range(4)], axis=0)
    w2t = jnp.flip(dcw0, axis=2).transpose(1, 0, 2)
    W2 = jnp.concatenate([_band(w2t[c, 0]) for c in range(4)], axis=1)
    b2 = jnp.repeat(dcb0, 64)
    w3t = jnp.flip(dcw1, axis=2).transpose(1, 0, 2)
    W3 = jnp.concatenate([_band(w3t[0, c]) for c in range(4)], axis=0)

    pa = _sc_aggr(x_pad, src2, dst2, ew2, zeros_f, n_pad, f)
    z1, s1, s2 = _tc_combine(pa, x_pad, enc0_Wrel, enc0_brel, enc0_Wroot,
                             n, True)
    z1bn = _tc_norm_relu(z1, s1, s2, ebn0_g, ebn0_b, n)
    pb = _sc_aggr(z1bn, src2, dst2, ew2, zeros_f, n_pad, f)
    zd, zd2 = _tc_enc1_cnn(pb, z1bn, enc1_Wrel, enc1_brel, enc1_Wroot,
                           W0, b0, ecb1, W1, W2, b2, dcb1, W3, dec0_Wrel)
    pc = _sc_aggr(zd2, src2, dst2, ew2, zeros_f, n_pad, f)
    h1, t1, t2 = _tc_combine(pc, zd, None, dec0_brel, dec0_Wroot,
                             n, True)
    h1bn = _tc_norm_relu(h1, t1, t2, dbn0_g, dbn0_b, n)
    pd = _sc_aggr(h1bn, src2, dst2, ew2, zeros_f, n_pad, f)
    hh = _tc_combine(pd, h1bn, dec1_Wrel, dec1_brel, dec1_Wroot, n, False)

    x_out = jnp.concatenate([x1, x2, x3, x4, x5, x6, x7], axis=1)
    h_out = hh[:n].reshape(b, 7 * f)
    return (x_out, h_out)
